# baseline (speedup 1.0000x reference)
import functools

import jax
import jax.numpy as jnp
from jax import lax
from jax.experimental import pallas as pl
from jax.experimental.pallas import tpu as pltpu
from jax.experimental.pallas import tpu_sc as plsc

LATENT_DIM = 64
LANES = 16
CHUNK = 256


def _make_mf_kernel(batch, num_workers, nc):
    b_per_w = batch // num_workers
    n_chunks = b_per_w // CHUNK
    mesh = plsc.VectorSubcoreMesh(core_axis_name="c", subcore_axis_name="s")

    @functools.partial(
        pl.kernel,
        mesh=mesh,
        out_type=jax.ShapeDtypeStruct((batch,), jnp.float32),
        scratch_types=[
            pltpu.VMEM((b_per_w,), jnp.int32),
            pltpu.VMEM((b_per_w,), jnp.int32),
            pltpu.VMEM((b_per_w,), jnp.int32),
            pltpu.VMEM((b_per_w,), jnp.int32),
            pltpu.VMEM((CHUNK, 2 * LATENT_DIM), jnp.float32),
            pltpu.VMEM((CHUNK, 2 * LATENT_DIM), jnp.float32),
            pltpu.VMEM((b_per_w,), jnp.float32),
            pltpu.SemaphoreType.DMA,
            pltpu.SemaphoreType.DMA,
        ],
        compiler_params=pltpu.CompilerParams(needs_layout_passes=False),
    )
    def mf(users_hbm, items_hbm, utab_hbm, itab_hbm, out_hbm,
           uidx_v, iidx_v, ust_v, ist_v, urows_v, irows_v, out_v,
           sem_u, sem_i):
        wid = lax.axis_index("s") * nc + lax.axis_index("c")
        base = wid * b_per_w

        pltpu.sync_copy(users_hbm.at[pl.ds(base, b_per_w)], uidx_v)
        pltpu.sync_copy(items_hbm.at[pl.ds(base, b_per_w)], iidx_v)

        def st_body(j, carry):
            sl = pl.ds(j * LANES, LANES)
            ust_v[sl] = lax.shift_right_logical(uidx_v[sl], 1)
            ist_v[sl] = lax.shift_right_logical(iidx_v[sl], 1)
            return carry

        lax.fori_loop(0, b_per_w // LANES, st_body, 0)

        lane_ids = lax.iota(jnp.int32, LANES)

        def chunk_body(c, carry):
            cbase = c * CHUNK
            cu = pltpu.async_copy(
                utab_hbm.at[ust_v.at[pl.ds(cbase, CHUNK)]], urows_v, sem_u)
            ci = pltpu.async_copy(
                itab_hbm.at[ist_v.at[pl.ds(cbase, CHUNK)]], irows_v, sem_i)
            cu.wait()
            ci.wait()

            def group_body(g, carry2):
                rows = g * LANES + lane_ids
                uofs = lax.shift_left(
                    jnp.bitwise_and(uidx_v[pl.ds(cbase + g * LANES, LANES)],
                                    1), 6)
                iofs = lax.shift_left(
                    jnp.bitwise_and(iidx_v[pl.ds(cbase + g * LANES, LANES)],
                                    1), 6)
                acc = jnp.zeros((LANES,), jnp.float32)
                for d in range(LATENT_DIM):
                    uv = plsc.load_gather(urows_v, [rows, uofs + d])
                    iv = plsc.load_gather(irows_v, [rows, iofs + d])
                    acc = acc + uv * iv
                out_v[pl.ds(cbase + g * LANES, LANES)] = (
                    1.0 / (1.0 + jnp.exp(-acc)))
                return carry2

            lax.fori_loop(0, CHUNK // LANES, group_body, 0)
            return carry

        lax.fori_loop(0, n_chunks, chunk_body, 0)
        pltpu.sync_copy(out_v, out_hbm.at[pl.ds(base, b_per_w)])

    return mf


def kernel(users, items, embedding_user, embedding_item):
    info = plsc.get_sparse_core_info()
    num_workers = info.num_cores * info.num_subcores
    mf = _make_mf_kernel(users.shape[0], num_workers, info.num_cores)
    nu = embedding_user.shape[0]
    ni = embedding_item.shape[0]
    ut2 = embedding_user.reshape(nu // 2, 2 * LATENT_DIM)
    it2 = embedding_item.reshape(ni // 2, 2 * LATENT_DIM)
    return mf(users.astype(jnp.int32), items.astype(jnp.int32), ut2, it2)
